# baseline (speedup 1.0000x reference)
import functools
import math

import jax
import jax.numpy as jnp
from jax import lax
from jax.experimental import pallas as pl
from jax.experimental.pallas import tpu as pltpu
from jax.experimental.pallas import tpu_sc as plsc

_D = 64
_V = 1000000
_SCALE = math.sqrt(_D)
_NC, _NS = 2, 16
_NW = _NC * _NS
_LANES = 128

_NFULL = _V // _LANES
_TAIL = _V - _NFULL * _LANES
_TAILW = _NFULL % _NW

_CH = 256
_IDXW = 128
_CHK = _CH // _IDXW


def _transpose_block(src_v, out_v, n_t2rows):
  iota = lax.iota(jnp.int32, 16)
  rows_half = iota >> 1
  colpar = (iota & 1) * _D
  ncols = n_t2rows * 2

  @plsc.parallel_loop(0, _D, 1, unroll=2)
  def _(d):
    cols = colpar + d
    for c0 in range(0, ncols, 16):
      vals = src_v[d, pl.ds(c0, 16)]
      plsc.store_scatter(out_v, [rows_half + (c0 // 2), cols], vals)


def _ph1_body(lut_t_hbm, tail_t_hbm, t2_hbm, src_a, src_b, tail_v,
              out_a, out_b, sem_ia, sem_ib, sem_oa, sem_ob):
  wid = lax.axis_index("s") * _NC + lax.axis_index("c")
  srcs = (src_a, src_b)
  outs = (out_a, out_b)
  sem_i = (sem_ia, sem_ib)
  sem_o = (sem_oa, sem_ob)
  n_i = (_NFULL - wid + _NW - 1) // _NW

  def fire_in(i, b):
    vb = wid + i * _NW
    pltpu.async_copy(lut_t_hbm.at[:, pl.ds(vb * _LANES, _LANES)],
                     srcs[b], sem_i[b])

  def wait_in(b):
    pltpu.make_async_copy(lut_t_hbm.at[:, pl.ds(0, _LANES)],
                          srcs[b], sem_i[b]).wait()

  def fire_out(i, b):
    vb = wid + i * _NW
    pltpu.async_copy(outs[b],
                     t2_hbm.at[pl.ds(vb * (_LANES // 2), _LANES // 2)],
                     sem_o[b])

  def wait_out(b):
    pltpu.make_async_copy(t2_hbm.at[pl.ds(0, _LANES // 2)],
                          outs[b], sem_o[b]).wait()

  fire_in(0, 0)

  @pl.loop(0, n_i, step=2)
  def _(ibase):
    for b in range(2):
      i = ibase + b

      @pl.when(i < n_i)
      def _():
        @pl.when(i + 1 < n_i)
        def _():
          fire_in(i + 1, 1 - b)

        wait_in(b)

        @pl.when(i >= 2)
        def _():
          wait_out(b)

        _transpose_block(srcs[b], outs[b], _LANES // 2)
        fire_out(i, b)

  wait_out(0)
  wait_out(1)

  @pl.when(wid == _TAILW)
  def _():
    pltpu.sync_copy(tail_t_hbm, tail_v)
    _transpose_block(tail_v, out_a, _TAIL // 2)
    pltpu.sync_copy(out_a.at[pl.ds(0, _TAIL // 2)],
                    t2_hbm.at[pl.ds(_NFULL * (_LANES // 2), _TAIL // 2)])


def _ph2_body(n_chunks, x_hbm, t2_hbm, out_hbm, idx_a, idx_b,
              pair_a, pair_b, off_a, off_b, rows_a, rows_b, out_st,
              sem_a, sem_b):
  wid = lax.axis_index("s") * _NC + lax.axis_index("c")
  base_idx_row = wid * n_chunks * _CHK
  base_out = wid * n_chunks * _CH

  idx_bufs = (idx_a, idx_b)
  pair_bufs = (pair_a, pair_b)
  off_bufs = (off_a, off_b)
  rows_bufs = (rows_a, rows_b)
  sems = (sem_a, sem_b)

  def stage_and_fire(g, b):
    pltpu.sync_copy(x_hbm.at[pl.ds(base_idx_row + g * _CHK, _CHK)],
                    idx_bufs[b])
    for k in range(_CHK):
      for s in range(_IDXW // 16):
        v = idx_bufs[b][k, pl.ds(s * 16, 16)]
        pair_bufs[b][k, pl.ds(s * 16, 16)] = v >> 1
        off_bufs[b][k, pl.ds(s * 16, 16)] = v & 1
    for k in range(_CHK):
      pltpu.async_copy(t2_hbm.at[pair_bufs[b].at[k]],
                       rows_bufs[b].at[pl.ds(k * _IDXW, _IDXW)], sems[b])

  def drain(b):
    for k in range(_CHK):
      pltpu.make_async_copy(t2_hbm.at[pl.ds(0, _IDXW)],
                            rows_bufs[b].at[pl.ds(k * _IDXW, _IDXW)],
                            sems[b]).wait()

  def select(b):
    kf = jnp.zeros((16,), jnp.int32)

    @plsc.parallel_loop(0, _CH // 16, 1, unroll=2)
    def _(gi):
      r0 = gi * 16
      for l in range(16):
        r = r0 + l
        par = plsc.load_gather(off_bufs[b],
                               [kf + (r // _IDXW), kf + (r % _IDXW)])
        msk = par > 0
        for g in range(_D // 16):
          left = rows_bufs[b][r, pl.ds(g * 16, 16)]
          right = rows_bufs[b][r, pl.ds(_D + g * 16, 16)]
          out_st[r, pl.ds(g * 16, 16)] = (
              jnp.where(msk, right, left) * _SCALE)

  stage_and_fire(0, 0)

  @pl.loop(0, n_chunks, step=2)
  def _(gbase):
    for b in range(2):
      g = gbase + b

      @pl.when(g + 1 < n_chunks)
      def _():
        stage_and_fire(g + 1, 1 - b)

      drain(b)
      select(b)
      pltpu.sync_copy(out_st, out_hbm.at[pl.ds(base_out + g * _CH, _CH)])


@jax.jit
def kernel(x, lut):
  n_total = x.shape[0] * x.shape[1]
  assert n_total % (_NW * _CH) == 0
  n_chunks = n_total // (_NW * _CH)
  x2d = x.reshape(n_total // _IDXW, _IDXW).astype(jnp.int32)
  lut_t = lut.T
  tail_t = lax.slice(lut, (_V - _TAIL, 0), (_V, _D)).T

  mesh = plsc.VectorSubcoreMesh(core_axis_name="c", subcore_axis_name="s",
                                num_cores=_NC, num_subcores=_NS)
  params = pltpu.CompilerParams(use_tc_tiling_on_sc=True,
                                needs_layout_passes=False)

  t2 = pl.kernel(
      _ph1_body,
      out_type=jax.ShapeDtypeStruct((_V // 2, _LANES), jnp.float32),
      mesh=mesh,
      compiler_params=params,
      scratch_types=[
          pltpu.VMEM((_D, _LANES), jnp.float32),
          pltpu.VMEM((_D, _LANES), jnp.float32),
          pltpu.VMEM((_D, _TAIL), jnp.float32),
          pltpu.VMEM((_LANES // 2, _LANES), jnp.float32),
          pltpu.VMEM((_LANES // 2, _LANES), jnp.float32),
          pltpu.SemaphoreType.DMA,
          pltpu.SemaphoreType.DMA,
          pltpu.SemaphoreType.DMA,
          pltpu.SemaphoreType.DMA,
      ],
  )(lut_t, tail_t)

  out = pl.kernel(
      functools.partial(_ph2_body, n_chunks),
      out_type=jax.ShapeDtypeStruct((n_total, _D), jnp.float32),
      mesh=mesh,
      compiler_params=params,
      scratch_types=[
          pltpu.VMEM((_CHK, _IDXW), jnp.int32),
          pltpu.VMEM((_CHK, _IDXW), jnp.int32),
          pltpu.VMEM((_CHK, _IDXW), jnp.int32),
          pltpu.VMEM((_CHK, _IDXW), jnp.int32),
          pltpu.VMEM((_CHK, _IDXW), jnp.int32),
          pltpu.VMEM((_CHK, _IDXW), jnp.int32),
          pltpu.VMEM((_CH, _LANES), jnp.float32),
          pltpu.VMEM((_CH, _LANES), jnp.float32),
          pltpu.VMEM((_CH, _D), jnp.float32),
          pltpu.SemaphoreType.DMA,
          pltpu.SemaphoreType.DMA,
      ],
  )(x2d, t2)
  return out.reshape(x.shape[0], x.shape[1], _D)


# speedup vs baseline: 1.0500x; 1.0500x over previous
import functools
import math

import jax
import jax.numpy as jnp
from jax import lax
from jax.experimental import pallas as pl
from jax.experimental.pallas import tpu as pltpu
from jax.experimental.pallas import tpu_sc as plsc

_D = 64
_V = 1000000
_SCALE = math.sqrt(_D)
_NC, _NS = 2, 16
_NW = _NC * _NS
_LANES = 128

_NFULL = _V // _LANES
_TAIL = _V - _NFULL * _LANES
_TAILW = _NFULL % _NW

_CH = 256
_IDXW = 128
_CHK = _CH // _IDXW


def _transpose_block(src_v, out_v, n_t2rows):
  d_ids = [lax.iota(jnp.int32, 16) + (g * 16) for g in range(_D // 16)]
  kf = jnp.zeros((16,), jnp.int32)

  @plsc.parallel_loop(0, n_t2rows, 1, unroll=8)
  def _(p):
    col_even = kf + 2 * p
    for q in range(2):
      col = col_even + q
      for g in range(_D // 16):
        vals = plsc.load_gather(src_v, [d_ids[g], col])
        out_v[p, pl.ds(q * _D + g * 16, 16)] = vals


def _ph1_body(lut_t_hbm, tail_t_hbm, t2_hbm, src_a, src_b, tail_v,
              out_a, out_b, sem_ia, sem_ib, sem_oa, sem_ob):
  wid = lax.axis_index("s") * _NC + lax.axis_index("c")
  srcs = (src_a, src_b)
  outs = (out_a, out_b)
  sem_i = (sem_ia, sem_ib)
  sem_o = (sem_oa, sem_ob)
  n_i = (_NFULL - wid + _NW - 1) // _NW

  def fire_in(i, b):
    vb = wid + i * _NW
    pltpu.async_copy(lut_t_hbm.at[:, pl.ds(vb * _LANES, _LANES)],
                     srcs[b], sem_i[b])

  def wait_in(b):
    pltpu.make_async_copy(lut_t_hbm.at[:, pl.ds(0, _LANES)],
                          srcs[b], sem_i[b]).wait()

  def fire_out(i, b):
    vb = wid + i * _NW
    pltpu.async_copy(outs[b],
                     t2_hbm.at[pl.ds(vb * (_LANES // 2), _LANES // 2)],
                     sem_o[b])

  def wait_out(b):
    pltpu.make_async_copy(t2_hbm.at[pl.ds(0, _LANES // 2)],
                          outs[b], sem_o[b]).wait()

  fire_in(0, 0)

  @pl.loop(0, n_i, step=2)
  def _(ibase):
    for b in range(2):
      i = ibase + b

      @pl.when(i < n_i)
      def _():
        @pl.when(i + 1 < n_i)
        def _():
          fire_in(i + 1, 1 - b)

        wait_in(b)

        @pl.when(i >= 2)
        def _():
          wait_out(b)

        _transpose_block(srcs[b], outs[b], _LANES // 2)
        fire_out(i, b)

  wait_out(0)
  wait_out(1)

  @pl.when(wid == _TAILW)
  def _():
    pltpu.sync_copy(tail_t_hbm, tail_v)
    _transpose_block(tail_v, out_a, _TAIL // 2)
    pltpu.sync_copy(out_a.at[pl.ds(0, _TAIL // 2)],
                    t2_hbm.at[pl.ds(_NFULL * (_LANES // 2), _TAIL // 2)])


def _ph2_body(n_chunks, x_hbm, t2_hbm, out_hbm, idx_a, idx_b,
              pair_a, pair_b, off_a, off_b, rows_a, rows_b, out_st,
              sem_a, sem_b):
  wid = lax.axis_index("s") * _NC + lax.axis_index("c")
  base_idx_row = wid * n_chunks * _CHK
  base_out = wid * n_chunks * _CH

  idx_bufs = (idx_a, idx_b)
  pair_bufs = (pair_a, pair_b)
  off_bufs = (off_a, off_b)
  rows_bufs = (rows_a, rows_b)
  sems = (sem_a, sem_b)

  def stage_and_fire(g, b):
    pltpu.sync_copy(x_hbm.at[pl.ds(base_idx_row + g * _CHK, _CHK)],
                    idx_bufs[b])
    for k in range(_CHK):
      for s in range(_IDXW // 16):
        v = idx_bufs[b][k, pl.ds(s * 16, 16)]
        pair_bufs[b][k, pl.ds(s * 16, 16)] = v >> 1
        off_bufs[b][k, pl.ds(s * 16, 16)] = v & 1
    for k in range(_CHK):
      pltpu.async_copy(t2_hbm.at[pair_bufs[b].at[k]],
                       rows_bufs[b].at[pl.ds(k * _IDXW, _IDXW)], sems[b])

  def drain(b):
    for k in range(_CHK):
      pltpu.make_async_copy(t2_hbm.at[pl.ds(0, _IDXW)],
                            rows_bufs[b].at[pl.ds(k * _IDXW, _IDXW)],
                            sems[b]).wait()

  def select(b):
    kf = jnp.zeros((16,), jnp.int32)

    @plsc.parallel_loop(0, _CH // 16, 1, unroll=2)
    def _(gi):
      r0 = gi * 16
      for l in range(16):
        r = r0 + l
        par = plsc.load_gather(off_bufs[b],
                               [kf + (r // _IDXW), kf + (r % _IDXW)])
        msk = par > 0
        for g in range(_D // 16):
          left = rows_bufs[b][r, pl.ds(g * 16, 16)]
          right = rows_bufs[b][r, pl.ds(_D + g * 16, 16)]
          out_st[r, pl.ds(g * 16, 16)] = (
              jnp.where(msk, right, left) * _SCALE)

  stage_and_fire(0, 0)

  @pl.loop(0, n_chunks, step=2)
  def _(gbase):
    for b in range(2):
      g = gbase + b

      @pl.when(g + 1 < n_chunks)
      def _():
        stage_and_fire(g + 1, 1 - b)

      drain(b)
      select(b)
      pltpu.sync_copy(out_st, out_hbm.at[pl.ds(base_out + g * _CH, _CH)])


@jax.jit
def kernel(x, lut):
  n_total = x.shape[0] * x.shape[1]
  assert n_total % (_NW * _CH) == 0
  n_chunks = n_total // (_NW * _CH)
  x2d = x.reshape(n_total // _IDXW, _IDXW).astype(jnp.int32)
  lut_t = lut.T
  tail_t = lax.slice(lut, (_V - _TAIL, 0), (_V, _D)).T

  mesh = plsc.VectorSubcoreMesh(core_axis_name="c", subcore_axis_name="s",
                                num_cores=_NC, num_subcores=_NS)
  params = pltpu.CompilerParams(use_tc_tiling_on_sc=True,
                                needs_layout_passes=False)

  t2 = pl.kernel(
      _ph1_body,
      out_type=jax.ShapeDtypeStruct((_V // 2, _LANES), jnp.float32),
      mesh=mesh,
      compiler_params=params,
      scratch_types=[
          pltpu.VMEM((_D, _LANES), jnp.float32),
          pltpu.VMEM((_D, _LANES), jnp.float32),
          pltpu.VMEM((_D, _TAIL), jnp.float32),
          pltpu.VMEM((_LANES // 2, _LANES), jnp.float32),
          pltpu.VMEM((_LANES // 2, _LANES), jnp.float32),
          pltpu.SemaphoreType.DMA,
          pltpu.SemaphoreType.DMA,
          pltpu.SemaphoreType.DMA,
          pltpu.SemaphoreType.DMA,
      ],
  )(lut_t, tail_t)

  out = pl.kernel(
      functools.partial(_ph2_body, n_chunks),
      out_type=jax.ShapeDtypeStruct((n_total, _D), jnp.float32),
      mesh=mesh,
      compiler_params=params,
      scratch_types=[
          pltpu.VMEM((_CHK, _IDXW), jnp.int32),
          pltpu.VMEM((_CHK, _IDXW), jnp.int32),
          pltpu.VMEM((_CHK, _IDXW), jnp.int32),
          pltpu.VMEM((_CHK, _IDXW), jnp.int32),
          pltpu.VMEM((_CHK, _IDXW), jnp.int32),
          pltpu.VMEM((_CHK, _IDXW), jnp.int32),
          pltpu.VMEM((_CH, _LANES), jnp.float32),
          pltpu.VMEM((_CH, _LANES), jnp.float32),
          pltpu.VMEM((_CH, _D), jnp.float32),
          pltpu.SemaphoreType.DMA,
          pltpu.SemaphoreType.DMA,
      ],
  )(x2d, t2)
  return out.reshape(x.shape[0], x.shape[1], _D)


# speedup vs baseline: 1.2298x; 1.1713x over previous
import functools
import math

import jax
import jax.numpy as jnp
from jax import lax
from jax.experimental import pallas as pl
from jax.experimental.pallas import tpu as pltpu
from jax.experimental.pallas import tpu_sc as plsc

_D = 64
_V = 1000000
_SCALE = math.sqrt(_D)
_NC, _NS = 2, 16
_NW = _NC * _NS
_LANES = 128

_CH = 256
_IDXW = 128
_CHK = _CH // _IDXW


def _body(n_chunks, x_hbm, t2_hbm, out_hbm, idx_a, idx_b,
          pair_a, pair_b, off_a, off_b, rows_a, rows_b, out_st,
          sem_a, sem_b):
  wid = lax.axis_index("s") * _NC + lax.axis_index("c")
  base_idx_row = wid * n_chunks * _CHK
  base_out = wid * n_chunks * _CH

  idx_bufs = (idx_a, idx_b)
  pair_bufs = (pair_a, pair_b)
  off_bufs = (off_a, off_b)
  rows_bufs = (rows_a, rows_b)
  sems = (sem_a, sem_b)

  def stage_and_fire(g, b):
    pltpu.sync_copy(x_hbm.at[pl.ds(base_idx_row + g * _CHK, _CHK)],
                    idx_bufs[b])
    for k in range(_CHK):
      for s in range(_IDXW // 16):
        v = idx_bufs[b][k, pl.ds(s * 16, 16)]
        pair_bufs[b][k, pl.ds(s * 16, 16)] = v >> 1
        off_bufs[b][k, pl.ds(s * 16, 16)] = v & 1
    for k in range(_CHK):
      pltpu.async_copy(t2_hbm.at[pair_bufs[b].at[k]],
                       rows_bufs[b].at[pl.ds(k * _IDXW, _IDXW)], sems[b])

  def drain(b):
    for k in range(_CHK):
      pltpu.make_async_copy(t2_hbm.at[pl.ds(0, _IDXW)],
                            rows_bufs[b].at[pl.ds(k * _IDXW, _IDXW)],
                            sems[b]).wait()

  def select(b):
    kf = jnp.zeros((16,), jnp.int32)

    @plsc.parallel_loop(0, _CH // 16, 1, unroll=2)
    def _(gi):
      r0 = gi * 16
      for l in range(16):
        r = r0 + l
        par = plsc.load_gather(off_bufs[b],
                               [kf + (r // _IDXW), kf + (r % _IDXW)])
        msk = par > 0
        for g in range(_D // 16):
          left = rows_bufs[b][r, pl.ds(g * 16, 16)]
          right = rows_bufs[b][r, pl.ds(_D + g * 16, 16)]
          out_st[r, pl.ds(g * 16, 16)] = (
              jnp.where(msk, right, left) * _SCALE)

  stage_and_fire(0, 0)

  @pl.loop(0, n_chunks, step=2)
  def _(gbase):
    for b in range(2):
      g = gbase + b

      @pl.when(g + 1 < n_chunks)
      def _():
        stage_and_fire(g + 1, 1 - b)

      drain(b)
      select(b)
      pltpu.sync_copy(out_st, out_hbm.at[pl.ds(base_out + g * _CH, _CH)])


@jax.jit
def kernel(x, lut):
  n_total = x.shape[0] * x.shape[1]
  assert n_total % (_NW * _CH) == 0
  n_chunks = n_total // (_NW * _CH)
  x2d = x.reshape(n_total // _IDXW, _IDXW).astype(jnp.int32)
  t2 = lut.reshape(_V // 2, _LANES)

  mesh = plsc.VectorSubcoreMesh(core_axis_name="c", subcore_axis_name="s",
                                num_cores=_NC, num_subcores=_NS)
  params = pltpu.CompilerParams(use_tc_tiling_on_sc=True,
                                needs_layout_passes=False)

  out = pl.kernel(
      functools.partial(_body, n_chunks),
      out_type=jax.ShapeDtypeStruct((n_total, _D), jnp.float32),
      mesh=mesh,
      compiler_params=params,
      scratch_types=[
          pltpu.VMEM((_CHK, _IDXW), jnp.int32),
          pltpu.VMEM((_CHK, _IDXW), jnp.int32),
          pltpu.VMEM((_CHK, _IDXW), jnp.int32),
          pltpu.VMEM((_CHK, _IDXW), jnp.int32),
          pltpu.VMEM((_CHK, _IDXW), jnp.int32),
          pltpu.VMEM((_CHK, _IDXW), jnp.int32),
          pltpu.VMEM((_CH, _LANES), jnp.float32),
          pltpu.VMEM((_CH, _LANES), jnp.float32),
          pltpu.VMEM((_CH, _D), jnp.float32),
          pltpu.SemaphoreType.DMA,
          pltpu.SemaphoreType.DMA,
      ],
  )(x2d, t2)
  return out.reshape(x.shape[0], x.shape[1], _D)
